# baseline (speedup 1.0000x reference)
import jax
import jax.numpy as jnp
from jax.experimental import pallas as pl

_L = 8
_K = 1024
_CD = 16
_EPS = 1e-12
_HALVES = 2


def _bf(x):
    return x.astype(jnp.bfloat16)


def _level_stage(res, i, wdt_ref, wut_ref, bu_ref, bdrow_ref, cbk_ref,
                 cbn_aug, lane):
    f32 = jnp.float32
    blk = res.shape[0]
    lo = i * _CD
    hi = lo + _CD
    ze = jnp.dot(_bf(res), _bf(wdt_ref[:, lo:hi]),
                 preferred_element_type=f32) + bdrow_ref[:, lo:hi]
    zf2 = jnp.sum(ze * ze, axis=1, keepdims=True)
    zf = ze / jnp.maximum(jnp.sqrt(zf2), _EPS)
    zfn2 = jnp.sum(zf * zf, axis=1, keepdims=True)
    zf_aug = jnp.concatenate(
        [-2.0 * zf, zfn2, jnp.ones((blk, 1), f32)], axis=1)
    d2 = jnp.dot(_bf(zf_aug), cbn_aug, preferred_element_type=f32)
    codes = jnp.argmin(d2, axis=1, keepdims=True).astype(jnp.int32)
    oh = (lane == codes).astype(jnp.bfloat16)
    zq = jnp.dot(oh, _bf(cbk_ref[:, lo:hi]), preferred_element_type=f32)
    up = jnp.dot(_bf(zq), _bf(wut_ref[lo:hi, :]), preferred_element_type=f32)
    new_res = res - (up + bu_ref[i:i + 1, :])
    d = ze - zq
    return new_res, ze + (zq - ze), codes, jnp.sum(d * d)


def _rvq_body(x_ref, wdt_ref, wut_ref, bu_ref, bdrow_ref, cbk_ref, cbt_ref,
              zq_ref, codes_ref, loss_ref):
    f32 = jnp.float32
    blk = x_ref.shape[0]
    h = blk // _HALVES
    res = [x_ref[k * h:(k + 1) * h, :] for k in range(_HALVES)]
    zq_cols = [[] for _ in range(_HALVES)]
    code_cols = [[] for _ in range(_HALVES)]
    loss_sum = jnp.zeros((), dtype=f32)
    lane = jax.lax.broadcasted_iota(jnp.int32, (h, _K), 1)
    for i in range(_L):
        lo = i * _CD
        hi = lo + _CD
        cbt_i = cbt_ref[lo:hi, :]
        cn = jnp.sqrt(jnp.sum(cbt_i * cbt_i, axis=0, keepdims=True))
        cbnt = cbt_i / jnp.maximum(cn, _EPS)
        cbn2 = jnp.sum(cbnt * cbnt, axis=0, keepdims=True)
        cbn_aug = jnp.concatenate(
            [_bf(cbnt), jnp.ones((1, _K), jnp.bfloat16), _bf(cbn2)], axis=0)
        for k in range(_HALVES):
            res[k], zq_st, codes, ls = _level_stage(
                res[k], i, wdt_ref, wut_ref, bu_ref, bdrow_ref, cbk_ref,
                cbn_aug, lane)
            zq_cols[k].append(zq_st)
            code_cols[k].append(codes)
            loss_sum = loss_sum + ls
    for k in range(_HALVES):
        sl = slice(k * h, (k + 1) * h)
        zq_ref[sl, :] = jnp.concatenate(zq_cols[k], axis=1)
        codes_ref[sl, :] = jnp.concatenate(code_cols[k], axis=1)
    loss_ref[0, :, :] = jnp.full((8, 128), loss_sum, dtype=f32)


def kernel(z_e, Wd, bd, cb, Wu, bu):
    Bc, Tc, Dc = z_e.shape
    n = Bc * Tc
    f32 = jnp.float32

    x = z_e.reshape(n, Dc)
    wdt = Wd.reshape(_L * _CD, Dc).T
    wut = Wu.transpose(0, 2, 1).reshape(_L * _CD, Dc)
    bdrow = bd.reshape(1, _L * _CD)
    cbk = cb.transpose(1, 0, 2).reshape(_K, _L * _CD)
    cbt2d = cb.transpose(0, 2, 1).reshape(_L * _CD, _K)

    blk = 2048
    nb = n // blk
    grid = (nb,)

    zq_out, codes_out, loss_out = pl.pallas_call(
        _rvq_body,
        grid=grid,
        in_specs=[
            pl.BlockSpec((blk, Dc), lambda b: (b, 0)),
            pl.BlockSpec((Dc, _L * _CD), lambda b: (0, 0)),
            pl.BlockSpec((_L * _CD, Dc), lambda b: (0, 0)),
            pl.BlockSpec((_L, Dc), lambda b: (0, 0)),
            pl.BlockSpec((1, _L * _CD), lambda b: (0, 0)),
            pl.BlockSpec((_K, _L * _CD), lambda b: (0, 0)),
            pl.BlockSpec((_L * _CD, _K), lambda b: (0, 0)),
        ],
        out_specs=[
            pl.BlockSpec((blk, _L * _CD), lambda b: (b, 0)),
            pl.BlockSpec((blk, _L), lambda b: (b, 0)),
            pl.BlockSpec((1, 8, 128), lambda b: (b, 0, 0)),
        ],
        out_shape=[
            jax.ShapeDtypeStruct((n, _L * _CD), f32),
            jax.ShapeDtypeStruct((n, _L), jnp.int32),
            jax.ShapeDtypeStruct((nb, 8, 128), f32),
        ],
    )(x, wdt, wut, bu, bdrow, cbk, cbt2d)

    z_q_concat = zq_out.reshape(Bc, Tc, _L * _CD)
    codes = codes_out.reshape(Bc, Tc, _L)
    total = jnp.sum(loss_out[:, 0, 0])
    commit = total / jnp.asarray(n * _CD, dtype=f32)
    cb_loss = total / jnp.asarray(n * _CD, dtype=f32)
    entropy_loss = jnp.zeros((), dtype=f32)
    return (z_q_concat, codes, commit, cb_loss, entropy_loss)


# speedup vs baseline: 1.2630x; 1.2630x over previous
import jax
import jax.numpy as jnp
from jax.experimental import pallas as pl

_L = 8
_K = 1024
_CD = 16
_EPS = 1e-12
_HALVES = 1


def _bf(x):
    return x.astype(jnp.bfloat16)


def _level_stage(res, i, wdt_ref, wut_ref, bu_ref, bdrow_ref, cbk_ref,
                 cbn_aug, hilo):
    f32 = jnp.float32
    blk = res.shape[0]
    lo = i * _CD
    hi = lo + _CD
    ze = jnp.dot(_bf(res), _bf(wdt_ref[:, lo:hi]),
                 preferred_element_type=f32) + bdrow_ref[:, lo:hi]
    zf2 = jnp.sum(ze * ze, axis=1, keepdims=True)
    zf = ze / jnp.maximum(jnp.sqrt(zf2), _EPS)
    zfn2 = jnp.sum(zf * zf, axis=1, keepdims=True)
    zf_aug = jnp.concatenate(
        [-2.0 * zf, zfn2, jnp.ones((blk, 1), f32)], axis=1)
    d2 = jnp.dot(_bf(zf_aug), cbn_aug, preferred_element_type=f32)
    minv = jnp.min(d2, axis=1, keepdims=True)
    oh = (d2 <= minv).astype(jnp.bfloat16)
    cb_aug = jnp.concatenate(
        [_bf(cbk_ref[:, lo:hi]), hilo], axis=1)
    zqc = jnp.dot(oh, cb_aug, preferred_element_type=f32)
    codes = (zqc[:, _CD:_CD + 1] * 32.0
             + zqc[:, _CD + 1:_CD + 2]).astype(jnp.int32)
    zq = zqc[:, :_CD]
    up = jnp.dot(_bf(zq), _bf(wut_ref[lo:hi, :]), preferred_element_type=f32)
    new_res = res - (up + bu_ref[i:i + 1, :])
    d = ze - zq
    return new_res, ze + (zq - ze), codes, jnp.sum(d * d)


def _rvq_body(x_ref, wdt_ref, wut_ref, bu_ref, bdrow_ref, cbk_ref, cbt_ref,
              zq_ref, codes_ref, loss_ref):
    f32 = jnp.float32
    blk = x_ref.shape[0]
    h = blk // _HALVES
    res = [x_ref[k * h:(k + 1) * h, :] for k in range(_HALVES)]
    zq_cols = [[] for _ in range(_HALVES)]
    code_cols = [[] for _ in range(_HALVES)]
    loss_sum = jnp.zeros((), dtype=f32)
    kio = jax.lax.broadcasted_iota(jnp.int32, (_K, 2), 0)
    hilo = jnp.where(
        jax.lax.broadcasted_iota(jnp.int32, (_K, 2), 1) == 0,
        kio // 32, kio % 32).astype(jnp.bfloat16)
    for i in range(_L):
        lo = i * _CD
        hi = lo + _CD
        cbt_i = cbt_ref[lo:hi, :]
        cn = jnp.sqrt(jnp.sum(cbt_i * cbt_i, axis=0, keepdims=True))
        cbnt = cbt_i / jnp.maximum(cn, _EPS)
        cbn2 = jnp.sum(cbnt * cbnt, axis=0, keepdims=True)
        cbn_aug = jnp.concatenate(
            [_bf(cbnt), jnp.ones((1, _K), jnp.bfloat16), _bf(cbn2)], axis=0)
        for k in range(_HALVES):
            res[k], zq_st, codes, ls = _level_stage(
                res[k], i, wdt_ref, wut_ref, bu_ref, bdrow_ref, cbk_ref,
                cbn_aug, hilo)
            zq_cols[k].append(zq_st)
            code_cols[k].append(codes)
            loss_sum = loss_sum + ls
    for k in range(_HALVES):
        sl = slice(k * h, (k + 1) * h)
        zq_ref[sl, :] = jnp.concatenate(zq_cols[k], axis=1)
        codes_ref[sl, :] = jnp.concatenate(code_cols[k], axis=1)
    loss_ref[0, :, :] = jnp.full((8, 128), loss_sum, dtype=f32)


def kernel(z_e, Wd, bd, cb, Wu, bu):
    Bc, Tc, Dc = z_e.shape
    n = Bc * Tc
    f32 = jnp.float32

    x = z_e.reshape(n, Dc)
    wdt = Wd.reshape(_L * _CD, Dc).T
    wut = Wu.transpose(0, 2, 1).reshape(_L * _CD, Dc)
    bdrow = bd.reshape(1, _L * _CD)
    cbk = cb.transpose(1, 0, 2).reshape(_K, _L * _CD)
    cbt2d = cb.transpose(0, 2, 1).reshape(_L * _CD, _K)

    blk = 1024
    nb = n // blk
    grid = (nb,)

    zq_out, codes_out, loss_out = pl.pallas_call(
        _rvq_body,
        grid=grid,
        in_specs=[
            pl.BlockSpec((blk, Dc), lambda b: (b, 0)),
            pl.BlockSpec((Dc, _L * _CD), lambda b: (0, 0)),
            pl.BlockSpec((_L * _CD, Dc), lambda b: (0, 0)),
            pl.BlockSpec((_L, Dc), lambda b: (0, 0)),
            pl.BlockSpec((1, _L * _CD), lambda b: (0, 0)),
            pl.BlockSpec((_K, _L * _CD), lambda b: (0, 0)),
            pl.BlockSpec((_L * _CD, _K), lambda b: (0, 0)),
        ],
        out_specs=[
            pl.BlockSpec((blk, _L * _CD), lambda b: (b, 0)),
            pl.BlockSpec((blk, _L), lambda b: (b, 0)),
            pl.BlockSpec((1, 8, 128), lambda b: (b, 0, 0)),
        ],
        out_shape=[
            jax.ShapeDtypeStruct((n, _L * _CD), f32),
            jax.ShapeDtypeStruct((n, _L), jnp.int32),
            jax.ShapeDtypeStruct((nb, 8, 128), f32),
        ],
    )(x, wdt, wut, bu, bdrow, cbk, cbt2d)

    z_q_concat = zq_out.reshape(Bc, Tc, _L * _CD)
    codes = codes_out.reshape(Bc, Tc, _L)
    total = jnp.sum(loss_out[:, 0, 0])
    commit = total / jnp.asarray(n * _CD, dtype=f32)
    cb_loss = total / jnp.asarray(n * _CD, dtype=f32)
    entropy_loss = jnp.zeros((), dtype=f32)
    return (z_q_concat, codes, commit, cb_loss, entropy_loss)
